# baseline (speedup 1.0000x reference)
import jax
import jax.numpy as jnp
import numpy as np
from jax import lax
from jax.experimental import pallas as pl
from jax.experimental.pallas import tpu as pltpu
from jax.experimental.pallas import tpu_sc as plsc

_N = 10000
_E = 320000
_D = 128
_HOR = 24

_NC = 2
_NS = 16
_NW = _NC * _NS
_L = 16

_EPT = _E // _NW
_K = 80
_C = _EPT // _K
_RPT = _N // _NS
_NBUF = 3
_NSB = 2

_SIGMA = np.concatenate([
    b0 + np.concatenate([2 * np.arange(16), 2 * np.arange(16) + 1])
    for b0 in range(0, _D, 32)
])


def _spmm_body(x_hbm, src_hbm, dst_hbm, w_hbm, out_hbm,
               src_t, bbufs, fbufs, dstc, wc, gsems, ssems, agg_sh):
    c = lax.axis_index("c")
    s = lax.axis_index("s")
    wid = s * _NC + c

    pltpu.sync_copy(src_hbm.at[wid], src_t)

    def _zrow(i, carry):
        for d in range(_D // _L):
            fbufs[0][i, pl.ds(d * _L, _L)] = jnp.zeros((_L,), jnp.float32)
        return carry

    lax.fori_loop(0, _K, _zrow, 0)
    base = s * _RPT
    for j in range(_RPT // _K):
        pltpu.sync_copy(fbufs[0], agg_sh.at[pl.ds(base + j * _K, _K)])
    rem = _RPT % _K
    if rem:
        pltpu.sync_copy(fbufs[0].at[pl.ds(0, rem)],
                        agg_sh.at[pl.ds(base + (_RPT // _K) * _K, rem)])
    plsc.subcore_barrier()

    def _scale(bbuf, fbuf, wbuf):
        hmask = jnp.full((_L,), -65536, jnp.int32)

        def _group(g, carry2):
            w16 = wbuf[pl.ds(g * _L, _L)]
            for j in range(_L):
                e = g * _L + j
                wv = lax.broadcast(w16[j], (_L,))
                for q in range(_D // 32):
                    v = bbuf[e, pl.ds(q * _L, _L)]
                    lo = lax.bitcast_convert_type(
                        lax.shift_left(v, 16), jnp.float32)
                    hi = lax.bitcast_convert_type(
                        lax.bitwise_and(v, hmask), jnp.float32)
                    fbuf[e, pl.ds(q * 32, _L)] = lo * wv
                    fbuf[e, pl.ds(q * 32 + _L, _L)] = hi * wv
            return carry2

        lax.fori_loop(0, _K // _L, _group, 0)

    def _fire_gather(ci, b):
        pltpu.async_copy(x_hbm.at[src_t.at[ci]], bbufs[b], gsems[b])
        pltpu.async_copy(dst_hbm.at[wid, ci], dstc[b], gsems[b])
        pltpu.async_copy(w_hbm.at[wid, ci], wc[b], gsems[b])

    def _wait_gather(ci, b):
        pltpu.make_async_copy(x_hbm.at[src_t.at[ci]], bbufs[b],
                              gsems[b]).wait()
        pltpu.make_async_copy(dst_hbm.at[wid, ci], dstc[b], gsems[b]).wait()
        pltpu.make_async_copy(w_hbm.at[wid, ci], wc[b], gsems[b]).wait()

    def _wait_scatter(bf32, bdst):
        pltpu.make_async_copy(fbufs[bf32], agg_sh.at[dstc[bdst]],
                              ssems[bf32]).wait()

    for b in range(_NBUF):
        _fire_gather(b, b)

    def _chunk(i, carry):
        for b6 in range(_NBUF * _NSB):
            b3 = b6 % _NBUF
            b2 = b6 % _NSB

            @pl.when(i % (_NBUF * _NSB) == b6)
            def _():
                _wait_gather(i, b3)

                @pl.when(i >= _NSB)
                def _():
                    _wait_scatter(b2, (b6 + 1) % _NBUF)

                _scale(bbufs[b3], fbufs[b2], wc[b3])
                pltpu.async_copy(fbufs[b2], agg_sh.at[dstc[b3]], ssems[b2],
                                 add=True)

                @pl.when(i + _NBUF < _C)
                def _():
                    _fire_gather(i + _NBUF, b3)

        return carry

    lax.fori_loop(0, _C, _chunk, 0)

    for ci in range(_C - _NSB, _C):
        _wait_scatter(ci % _NSB, ci % _NBUF)


    plsc.subcore_barrier()
    pltpu.sync_copy(agg_sh.at[pl.ds(base, _RPT)],
                    out_hbm.at[c, pl.ds(base, _RPT)])


_spmm = pl.kernel(
    _spmm_body,
    out_type=jax.ShapeDtypeStruct((_NC, _N, _D), jnp.float32),
    mesh=plsc.VectorSubcoreMesh(core_axis_name="c", subcore_axis_name="s"),
    compiler_params=pltpu.CompilerParams(use_tc_tiling_on_sc=False),
    scratch_types=[
        pltpu.VMEM((_C, _K), jnp.int32),
        tuple(pltpu.VMEM((_K, _D // 2), jnp.int32) for _ in range(_NBUF)),
        tuple(pltpu.VMEM((_K, _D), jnp.float32) for _ in range(_NSB)),
        tuple(pltpu.VMEM((_K,), jnp.int32) for _ in range(_NBUF)),
        tuple(pltpu.VMEM((_K,), jnp.float32) for _ in range(_NBUF)),
        tuple(pltpu.SemaphoreType.DMA for _ in range(_NBUF)),
        tuple(pltpu.SemaphoreType.DMA for _ in range(_NSB)),
        pltpu.VMEM_SHARED((_N, _D), jnp.float32),
    ],
)


def _dense_body(p_ref, x_ref, wrel_ref, wroot_ref, b_ref, o_ref, obf_ref):
    agg = p_ref[0] + p_ref[1]
    h = (jnp.dot(agg, wrel_ref[...], preferred_element_type=jnp.float32)
         + jnp.dot(x_ref[...], wroot_ref[...], preferred_element_type=jnp.float32)
         + b_ref[...])
    h = jnp.maximum(h, 0.0)
    o_ref[...] = h
    obf_ref[...] = h.astype(jnp.bfloat16)


def _head_body(p_ref, x_ref, wrel_ref, wroot_ref, b_ref, wfc_ref, bfc_ref,
               wlast_ref, blast_ref, o_ref):
    agg = p_ref[0] + p_ref[1]
    h = (jnp.dot(agg, wrel_ref[...], preferred_element_type=jnp.float32)
         + jnp.dot(x_ref[...], wroot_ref[...], preferred_element_type=jnp.float32)
         + b_ref[...])
    h = jnp.maximum(h, 0.0)
    h = jnp.maximum(
        jnp.dot(h, wfc_ref[...], preferred_element_type=jnp.float32) + bfc_ref[...],
        0.0)
    o_ref[...] = (jnp.dot(h, wlast_ref[...], preferred_element_type=jnp.float32)
                  + blast_ref[...])


_RB = 1000


def _dense(parts, x, w_rel, w_root, b):
    grid = (_N // _RB,)
    return pl.pallas_call(
        _dense_body,
        grid=grid,
        in_specs=[
            pl.BlockSpec((_NC, _RB, _D), lambda i: (0, i, 0)),
            pl.BlockSpec((_RB, _D), lambda i: (i, 0)),
            pl.BlockSpec((_D, _D), lambda i: (0, 0)),
            pl.BlockSpec((_D, _D), lambda i: (0, 0)),
            pl.BlockSpec((1, _D), lambda i: (0, 0)),
        ],
        out_specs=[
            pl.BlockSpec((_RB, _D), lambda i: (i, 0)),
            pl.BlockSpec((_RB, _D), lambda i: (i, 0)),
        ],
        out_shape=[
            jax.ShapeDtypeStruct((_N, _D), jnp.float32),
            jax.ShapeDtypeStruct((_N, _D), jnp.bfloat16),
        ],
    )(parts, x, w_rel, w_root, b.reshape(1, _D))


def _head(parts, x, w_rel, w_root, b, w_fc, b_fc, w_last_p, b_last_p):
    grid = (_N // _RB,)
    return pl.pallas_call(
        _head_body,
        grid=grid,
        in_specs=[
            pl.BlockSpec((_NC, _RB, _D), lambda i: (0, i, 0)),
            pl.BlockSpec((_RB, _D), lambda i: (i, 0)),
            pl.BlockSpec((_D, _D), lambda i: (0, 0)),
            pl.BlockSpec((_D, _D), lambda i: (0, 0)),
            pl.BlockSpec((1, _D), lambda i: (0, 0)),
            pl.BlockSpec((_D, _D), lambda i: (0, 0)),
            pl.BlockSpec((1, _D), lambda i: (0, 0)),
            pl.BlockSpec((_D, _D), lambda i: (0, 0)),
            pl.BlockSpec((1, _D), lambda i: (0, 0)),
        ],
        out_specs=pl.BlockSpec((_RB, _D), lambda i: (i, 0)),
        out_shape=jax.ShapeDtypeStruct((_N, _D), jnp.float32),
    )(parts, x, w_rel, w_root, b.reshape(1, _D), w_fc, b_fc.reshape(1, _D),
      w_last_p, b_last_p)


def kernel(x, edge_index, edge_weights, W_rel1, b_rel1, W_root1,
           W_rel2, b_rel2, W_root2, W_fc, b_fc, W_last, b_last):
    src = edge_index[0].astype(jnp.int32).reshape(_NW, _C, _K)
    dst = edge_index[1].astype(jnp.int32).reshape(_NW, _C, _K)
    w = edge_weights.reshape(_NW, _C, _K)

    sig = jnp.asarray(_SIGMA)
    W_rel1_s = W_rel1[sig, :]
    W_rel2_s = W_rel2[sig, :]

    w_last_p = jnp.zeros((_D, _D), jnp.float32).at[:, :_HOR].set(W_last)
    b_last_p = jnp.zeros((1, _D), jnp.float32).at[0, :_HOR].set(b_last)

    def _pack(a_bf):
        return lax.bitcast_convert_type(
            a_bf.reshape(_N, _D // 2, 2), jnp.int32)

    p1 = _spmm(_pack(x.astype(jnp.bfloat16)), src, dst, w)
    h1, h1_bf = _dense(p1, x, W_rel1_s, W_root1, b_rel1)
    p2 = _spmm(_pack(h1_bf), src, dst, w)
    h2, h2_bf = _dense(p2, h1, W_rel2_s, W_root2, b_rel2)
    p3 = _spmm(_pack(h2_bf), src, dst, w)
    out = _head(p3, h2, W_rel2_s, W_root2, b_rel2, W_fc, b_fc,
                w_last_p, b_last_p)
    return out[:, :_HOR]


# speedup vs baseline: 1.0431x; 1.0431x over previous
import jax
import jax.numpy as jnp
import numpy as np
from jax import lax
from jax.experimental import pallas as pl
from jax.experimental.pallas import tpu as pltpu
from jax.experimental.pallas import tpu_sc as plsc

_N = 10000
_E = 320000
_D = 128
_HOR = 24

_NC = 2
_NS = 16
_NW = _NC * _NS
_L = 16

_EPT = _E // _NW
_K = 80
_C = _EPT // _K
_RPT = _N // _NS
_NBUF = 2
_NSB = 2

_SIGMA = np.concatenate([
    b0 + np.concatenate([2 * np.arange(16), 2 * np.arange(16) + 1])
    for b0 in range(0, _D, 32)
])


def _spmm_body(x_hbm, src_hbm, dst_hbm, w_hbm, out_hbm,
               src_t, bbufs, fbufs, dstc, wc, gsems, ssems, agg_sh):
    c = lax.axis_index("c")
    s = lax.axis_index("s")
    wid = s * _NC + c

    pltpu.sync_copy(src_hbm.at[wid], src_t)

    def _zrow(i, carry):
        for d in range(_D // _L):
            fbufs[0][i, pl.ds(d * _L, _L)] = jnp.zeros((_L,), jnp.float32)
        return carry

    lax.fori_loop(0, _K, _zrow, 0)
    base = s * _RPT
    for j in range(_RPT // _K):
        pltpu.sync_copy(fbufs[0], agg_sh.at[pl.ds(base + j * _K, _K)])
    rem = _RPT % _K
    if rem:
        pltpu.sync_copy(fbufs[0].at[pl.ds(0, rem)],
                        agg_sh.at[pl.ds(base + (_RPT // _K) * _K, rem)])
    plsc.subcore_barrier()

    def _scale(bbuf, fbuf, wbuf):
        hmask = jnp.full((_L,), -65536, jnp.int32)

        def _group(g, carry2):
            w16 = wbuf[pl.ds(g * _L, _L)]
            for j in range(_L):
                e = g * _L + j
                wv = lax.broadcast(w16[j], (_L,))
                for q in range(_D // 32):
                    v = bbuf[e, pl.ds(q * _L, _L)]
                    lo = lax.bitcast_convert_type(
                        lax.shift_left(v, 16), jnp.float32)
                    hi = lax.bitcast_convert_type(
                        lax.bitwise_and(v, hmask), jnp.float32)
                    fbuf[e, pl.ds(q * 32, _L)] = lo * wv
                    fbuf[e, pl.ds(q * 32 + _L, _L)] = hi * wv
            return carry2

        lax.fori_loop(0, _K // _L, _group, 0)

    def _fire_gather(ci, b):
        pltpu.async_copy(x_hbm.at[src_t.at[ci]], bbufs[b], gsems[b])
        pltpu.async_copy(dst_hbm.at[wid, ci], dstc[b], gsems[b])
        pltpu.async_copy(w_hbm.at[wid, ci], wc[b], gsems[b])

    def _wait_gather(ci, b):
        pltpu.make_async_copy(x_hbm.at[src_t.at[ci]], bbufs[b],
                              gsems[b]).wait()
        pltpu.make_async_copy(dst_hbm.at[wid, ci], dstc[b], gsems[b]).wait()
        pltpu.make_async_copy(w_hbm.at[wid, ci], wc[b], gsems[b]).wait()

    def _wait_scatter(bf32, bdst):
        pltpu.make_async_copy(fbufs[bf32], agg_sh.at[dstc[bdst]],
                              ssems[bf32]).wait()

    for b in range(_NBUF):
        _fire_gather(b, b)

    def _chunk(i, carry):
        for b6 in range(_NBUF):
            b3 = b6 % _NBUF
            b2 = b6 % _NSB

            @pl.when(i % _NBUF == b6)
            def _():
                _wait_gather(i, b3)

                @pl.when(i >= _NSB)
                def _():
                    _wait_scatter(b2, (b6 + _NBUF - 2) % _NBUF)

                _scale(bbufs[b3], fbufs[b2], wc[b3])
                pltpu.async_copy(fbufs[b2], agg_sh.at[dstc[b3]], ssems[b2],
                                 add=True)

                @pl.when(i + _NBUF < _C)
                def _():
                    _fire_gather(i + _NBUF, b3)

        return carry

    lax.fori_loop(0, _C, _chunk, 0)

    for ci in range(_C - _NSB, _C):
        _wait_scatter(ci % _NSB, ci % _NBUF)


    plsc.subcore_barrier()
    pltpu.sync_copy(agg_sh.at[pl.ds(base, _RPT)],
                    out_hbm.at[c, pl.ds(base, _RPT)])


_spmm = pl.kernel(
    _spmm_body,
    out_type=jax.ShapeDtypeStruct((_NC, _N, _D), jnp.float32),
    mesh=plsc.VectorSubcoreMesh(core_axis_name="c", subcore_axis_name="s"),
    compiler_params=pltpu.CompilerParams(use_tc_tiling_on_sc=False),
    scratch_types=[
        pltpu.VMEM((_C, _K), jnp.int32),
        tuple(pltpu.VMEM((_K, _D // 2), jnp.int32) for _ in range(_NBUF)),
        tuple(pltpu.VMEM((_K, _D), jnp.float32) for _ in range(_NSB)),
        tuple(pltpu.VMEM((_K,), jnp.int32) for _ in range(_NBUF)),
        tuple(pltpu.VMEM((_K,), jnp.float32) for _ in range(_NBUF)),
        tuple(pltpu.SemaphoreType.DMA for _ in range(_NBUF)),
        tuple(pltpu.SemaphoreType.DMA for _ in range(_NSB)),
        pltpu.VMEM_SHARED((_N, _D), jnp.float32),
    ],
)


def _dense_body(p_ref, x_ref, wrel_ref, wroot_ref, b_ref, o_ref, obf_ref):
    agg = p_ref[0] + p_ref[1]
    h = (jnp.dot(agg, wrel_ref[...], preferred_element_type=jnp.float32)
         + jnp.dot(x_ref[...], wroot_ref[...], preferred_element_type=jnp.float32)
         + b_ref[...])
    h = jnp.maximum(h, 0.0)
    o_ref[...] = h
    obf_ref[...] = h.astype(jnp.bfloat16)


def _head_body(p_ref, x_ref, wrel_ref, wroot_ref, b_ref, wfc_ref, bfc_ref,
               wlast_ref, blast_ref, o_ref):
    agg = p_ref[0] + p_ref[1]
    h = (jnp.dot(agg, wrel_ref[...], preferred_element_type=jnp.float32)
         + jnp.dot(x_ref[...], wroot_ref[...], preferred_element_type=jnp.float32)
         + b_ref[...])
    h = jnp.maximum(h, 0.0)
    h = jnp.maximum(
        jnp.dot(h, wfc_ref[...], preferred_element_type=jnp.float32) + bfc_ref[...],
        0.0)
    o_ref[...] = (jnp.dot(h, wlast_ref[...], preferred_element_type=jnp.float32)
                  + blast_ref[...])


_RB = 1000


def _dense(parts, x, w_rel, w_root, b):
    grid = (_N // _RB,)
    return pl.pallas_call(
        _dense_body,
        grid=grid,
        in_specs=[
            pl.BlockSpec((_NC, _RB, _D), lambda i: (0, i, 0)),
            pl.BlockSpec((_RB, _D), lambda i: (i, 0)),
            pl.BlockSpec((_D, _D), lambda i: (0, 0)),
            pl.BlockSpec((_D, _D), lambda i: (0, 0)),
            pl.BlockSpec((1, _D), lambda i: (0, 0)),
        ],
        out_specs=[
            pl.BlockSpec((_RB, _D), lambda i: (i, 0)),
            pl.BlockSpec((_RB, _D), lambda i: (i, 0)),
        ],
        out_shape=[
            jax.ShapeDtypeStruct((_N, _D), jnp.float32),
            jax.ShapeDtypeStruct((_N, _D), jnp.bfloat16),
        ],
    )(parts, x, w_rel, w_root, b.reshape(1, _D))


def _head(parts, x, w_rel, w_root, b, w_fc, b_fc, w_last_p, b_last_p):
    grid = (_N // _RB,)
    return pl.pallas_call(
        _head_body,
        grid=grid,
        in_specs=[
            pl.BlockSpec((_NC, _RB, _D), lambda i: (0, i, 0)),
            pl.BlockSpec((_RB, _D), lambda i: (i, 0)),
            pl.BlockSpec((_D, _D), lambda i: (0, 0)),
            pl.BlockSpec((_D, _D), lambda i: (0, 0)),
            pl.BlockSpec((1, _D), lambda i: (0, 0)),
            pl.BlockSpec((_D, _D), lambda i: (0, 0)),
            pl.BlockSpec((1, _D), lambda i: (0, 0)),
            pl.BlockSpec((_D, _D), lambda i: (0, 0)),
            pl.BlockSpec((1, _D), lambda i: (0, 0)),
        ],
        out_specs=pl.BlockSpec((_RB, _D), lambda i: (i, 0)),
        out_shape=jax.ShapeDtypeStruct((_N, _D), jnp.float32),
    )(parts, x, w_rel, w_root, b.reshape(1, _D), w_fc, b_fc.reshape(1, _D),
      w_last_p, b_last_p)


def kernel(x, edge_index, edge_weights, W_rel1, b_rel1, W_root1,
           W_rel2, b_rel2, W_root2, W_fc, b_fc, W_last, b_last):
    src = edge_index[0].astype(jnp.int32).reshape(_NW, _C, _K)
    dst = edge_index[1].astype(jnp.int32).reshape(_NW, _C, _K)
    w = edge_weights.reshape(_NW, _C, _K)

    sig = jnp.asarray(_SIGMA)
    W_rel1_s = W_rel1[sig, :]
    W_rel2_s = W_rel2[sig, :]

    w_last_p = jnp.zeros((_D, _D), jnp.float32).at[:, :_HOR].set(W_last)
    b_last_p = jnp.zeros((1, _D), jnp.float32).at[0, :_HOR].set(b_last)

    def _pack(a_bf):
        return lax.bitcast_convert_type(
            a_bf.reshape(_N, _D // 2, 2), jnp.int32)

    p1 = _spmm(_pack(x.astype(jnp.bfloat16)), src, dst, w)
    h1, h1_bf = _dense(p1, x, W_rel1_s, W_root1, b_rel1)
    p2 = _spmm(_pack(h1_bf), src, dst, w)
    h2, h2_bf = _dense(p2, h1, W_rel2_s, W_root2, b_rel2)
    p3 = _spmm(_pack(h2_bf), src, dst, w)
    out = _head(p3, h2, W_rel2_s, W_root2, b_rel2, W_fc, b_fc,
                w_last_p, b_last_p)
    return out[:, :_HOR]


# speedup vs baseline: 1.2467x; 1.1951x over previous
import jax
import jax.numpy as jnp
import numpy as np
from jax import lax
from jax.experimental import pallas as pl
from jax.experimental.pallas import tpu as pltpu
from jax.experimental.pallas import tpu_sc as plsc

_N = 10000
_E = 320000
_D = 128
_HOR = 24

_NC = 2
_NS = 16
_NW = _NC * _NS
_L = 16

_EPT = _E // _NW
_K = 80
_C = _EPT // _K
_RPT = _N // _NS
_NBUF = 2
_NSB = 2

_SIGMA = np.concatenate([
    b0 + np.concatenate([2 * np.arange(16), 2 * np.arange(16) + 1])
    for b0 in range(0, _D, 32)
])


def _spmm_body(x_hbm, src_hbm, dst_hbm, w_hbm, out_hbm,
               src_t, bbufs, fbufs, dstc, wc, gsems, ssems, agg_sh):
    c = lax.axis_index("c")
    s = lax.axis_index("s")
    wid = s * _NC + c

    pltpu.sync_copy(src_hbm.at[wid], src_t)

    def _zrow(i, carry):
        for d in range(_D // _L):
            fbufs[0][i, pl.ds(d * _L, _L)] = jnp.zeros((_L,), jnp.float32)
        return carry

    lax.fori_loop(0, _K, _zrow, 0)
    base = s * _RPT
    for j in range(_RPT // _K):
        pltpu.sync_copy(fbufs[0], agg_sh.at[pl.ds(base + j * _K, _K)])
    rem = _RPT % _K
    if rem:
        pltpu.sync_copy(fbufs[0].at[pl.ds(0, rem)],
                        agg_sh.at[pl.ds(base + (_RPT // _K) * _K, rem)])
    plsc.subcore_barrier()

    def _scale(bbuf, fbuf, wbuf):
        hmask = jnp.full((_L,), -65536, jnp.int32)

        @plsc.parallel_loop(0, _K // _L, 1)
        def _group(g):
            w16 = wbuf[pl.ds(g * _L, _L)]
            for j in range(_L):
                e = g * _L + j
                wv = lax.broadcast(w16[j], (_L,))
                for q in range(_D // 32):
                    v = bbuf[e, pl.ds(q * _L, _L)]
                    lo = lax.bitcast_convert_type(
                        lax.shift_left(v, 16), jnp.float32)
                    hi = lax.bitcast_convert_type(
                        lax.bitwise_and(v, hmask), jnp.float32)
                    fbuf[e, pl.ds(q * 32, _L)] = lo * wv
                    fbuf[e, pl.ds(q * 32 + _L, _L)] = hi * wv

    def _fire_gather(ci, b):
        pltpu.async_copy(x_hbm.at[src_t.at[ci]], bbufs[b], gsems[b])
        pltpu.async_copy(dst_hbm.at[wid, ci], dstc[b], gsems[b])
        pltpu.async_copy(w_hbm.at[wid, ci], wc[b], gsems[b])

    def _wait_gather(ci, b):
        pltpu.make_async_copy(x_hbm.at[src_t.at[ci]], bbufs[b],
                              gsems[b]).wait()
        pltpu.make_async_copy(dst_hbm.at[wid, ci], dstc[b], gsems[b]).wait()
        pltpu.make_async_copy(w_hbm.at[wid, ci], wc[b], gsems[b]).wait()

    def _wait_scatter(bf32, bdst):
        pltpu.make_async_copy(fbufs[bf32], agg_sh.at[dstc[bdst]],
                              ssems[bf32]).wait()

    for b in range(_NBUF):
        _fire_gather(b, b)

    def _chunk(i, carry):
        for b6 in range(_NBUF):
            b3 = b6 % _NBUF
            b2 = b6 % _NSB

            @pl.when(i % _NBUF == b6)
            def _():
                _wait_gather(i, b3)

                @pl.when(i >= _NSB)
                def _():
                    _wait_scatter(b2, (b6 + _NBUF - 2) % _NBUF)

                _scale(bbufs[b3], fbufs[b2], wc[b3])
                pltpu.async_copy(fbufs[b2], agg_sh.at[dstc[b3]], ssems[b2],
                                 add=True)

                @pl.when(i + _NBUF < _C)
                def _():
                    _fire_gather(i + _NBUF, b3)

        return carry

    lax.fori_loop(0, _C, _chunk, 0)

    for ci in range(_C - _NSB, _C):
        _wait_scatter(ci % _NSB, ci % _NBUF)


    plsc.subcore_barrier()
    pltpu.sync_copy(agg_sh.at[pl.ds(base, _RPT)],
                    out_hbm.at[c, pl.ds(base, _RPT)])


_spmm = pl.kernel(
    _spmm_body,
    out_type=jax.ShapeDtypeStruct((_NC, _N, _D), jnp.float32),
    mesh=plsc.VectorSubcoreMesh(core_axis_name="c", subcore_axis_name="s"),
    compiler_params=pltpu.CompilerParams(use_tc_tiling_on_sc=False),
    scratch_types=[
        pltpu.VMEM((_C, _K), jnp.int32),
        tuple(pltpu.VMEM((_K, _D // 2), jnp.int32) for _ in range(_NBUF)),
        tuple(pltpu.VMEM((_K, _D), jnp.float32) for _ in range(_NSB)),
        tuple(pltpu.VMEM((_K,), jnp.int32) for _ in range(_NBUF)),
        tuple(pltpu.VMEM((_K,), jnp.float32) for _ in range(_NBUF)),
        tuple(pltpu.SemaphoreType.DMA for _ in range(_NBUF)),
        tuple(pltpu.SemaphoreType.DMA for _ in range(_NSB)),
        pltpu.VMEM_SHARED((_N, _D), jnp.float32),
    ],
)


def _dense_body(p_ref, x_ref, wrel_ref, wroot_ref, b_ref, o_ref, obf_ref):
    agg = p_ref[0] + p_ref[1]
    h = (jnp.dot(agg, wrel_ref[...], preferred_element_type=jnp.float32)
         + jnp.dot(x_ref[...], wroot_ref[...], preferred_element_type=jnp.float32)
         + b_ref[...])
    h = jnp.maximum(h, 0.0)
    o_ref[...] = h
    obf_ref[...] = h.astype(jnp.bfloat16)


def _head_body(p_ref, x_ref, wrel_ref, wroot_ref, b_ref, wfc_ref, bfc_ref,
               wlast_ref, blast_ref, o_ref):
    agg = p_ref[0] + p_ref[1]
    h = (jnp.dot(agg, wrel_ref[...], preferred_element_type=jnp.float32)
         + jnp.dot(x_ref[...], wroot_ref[...], preferred_element_type=jnp.float32)
         + b_ref[...])
    h = jnp.maximum(h, 0.0)
    h = jnp.maximum(
        jnp.dot(h, wfc_ref[...], preferred_element_type=jnp.float32) + bfc_ref[...],
        0.0)
    o_ref[...] = (jnp.dot(h, wlast_ref[...], preferred_element_type=jnp.float32)
                  + blast_ref[...])


_RB = 1000


def _dense(parts, x, w_rel, w_root, b):
    grid = (_N // _RB,)
    return pl.pallas_call(
        _dense_body,
        grid=grid,
        in_specs=[
            pl.BlockSpec((_NC, _RB, _D), lambda i: (0, i, 0)),
            pl.BlockSpec((_RB, _D), lambda i: (i, 0)),
            pl.BlockSpec((_D, _D), lambda i: (0, 0)),
            pl.BlockSpec((_D, _D), lambda i: (0, 0)),
            pl.BlockSpec((1, _D), lambda i: (0, 0)),
        ],
        out_specs=[
            pl.BlockSpec((_RB, _D), lambda i: (i, 0)),
            pl.BlockSpec((_RB, _D), lambda i: (i, 0)),
        ],
        out_shape=[
            jax.ShapeDtypeStruct((_N, _D), jnp.float32),
            jax.ShapeDtypeStruct((_N, _D), jnp.bfloat16),
        ],
    )(parts, x, w_rel, w_root, b.reshape(1, _D))


def _head(parts, x, w_rel, w_root, b, w_fc, b_fc, w_last_p, b_last_p):
    grid = (_N // _RB,)
    return pl.pallas_call(
        _head_body,
        grid=grid,
        in_specs=[
            pl.BlockSpec((_NC, _RB, _D), lambda i: (0, i, 0)),
            pl.BlockSpec((_RB, _D), lambda i: (i, 0)),
            pl.BlockSpec((_D, _D), lambda i: (0, 0)),
            pl.BlockSpec((_D, _D), lambda i: (0, 0)),
            pl.BlockSpec((1, _D), lambda i: (0, 0)),
            pl.BlockSpec((_D, _D), lambda i: (0, 0)),
            pl.BlockSpec((1, _D), lambda i: (0, 0)),
            pl.BlockSpec((_D, _D), lambda i: (0, 0)),
            pl.BlockSpec((1, _D), lambda i: (0, 0)),
        ],
        out_specs=pl.BlockSpec((_RB, _D), lambda i: (i, 0)),
        out_shape=jax.ShapeDtypeStruct((_N, _D), jnp.float32),
    )(parts, x, w_rel, w_root, b.reshape(1, _D), w_fc, b_fc.reshape(1, _D),
      w_last_p, b_last_p)


def kernel(x, edge_index, edge_weights, W_rel1, b_rel1, W_root1,
           W_rel2, b_rel2, W_root2, W_fc, b_fc, W_last, b_last):
    src = edge_index[0].astype(jnp.int32).reshape(_NW, _C, _K)
    dst = edge_index[1].astype(jnp.int32).reshape(_NW, _C, _K)
    w = edge_weights.reshape(_NW, _C, _K)

    sig = jnp.asarray(_SIGMA)
    W_rel1_s = W_rel1[sig, :]
    W_rel2_s = W_rel2[sig, :]

    w_last_p = jnp.zeros((_D, _D), jnp.float32).at[:, :_HOR].set(W_last)
    b_last_p = jnp.zeros((1, _D), jnp.float32).at[0, :_HOR].set(b_last)

    def _pack(a_bf):
        return lax.bitcast_convert_type(
            a_bf.reshape(_N, _D // 2, 2), jnp.int32)

    p1 = _spmm(_pack(x.astype(jnp.bfloat16)), src, dst, w)
    h1, h1_bf = _dense(p1, x, W_rel1_s, W_root1, b_rel1)
    p2 = _spmm(_pack(h1_bf), src, dst, w)
    h2, h2_bf = _dense(p2, h1, W_rel2_s, W_root2, b_rel2)
    p3 = _spmm(_pack(h2_bf), src, dst, w)
    out = _head(p3, h2, W_rel2_s, W_root2, b_rel2, W_fc, b_fc,
                w_last_p, b_last_p)
    return out[:, :_HOR]
